# Initial kernel scaffold; baseline (speedup 1.0000x reference)
import jax
import jax.numpy as jnp
from jax.experimental import pallas as pl


def kernel(x, inds, w, b):
    raise NotImplementedError("write your pallas kernel here")


# speedup vs baseline: 3.7270x; 3.7270x over previous
import jax
import jax.numpy as jnp
from jax.experimental import pallas as pl

_CLASSES = 64
_M = 256
_N = 256
_T = 2048


def _body(inds_ref, x_ref, w_ref, b_ref, o_ref):
    c = pl.program_id(0)
    y = jnp.dot(x_ref[...], w_ref[0], preferred_element_type=jnp.float32)
    y = y + b_ref[0]
    m = inds_ref[...] == c
    prev = jnp.where(c == 0, jnp.zeros_like(y), o_ref[...])
    o_ref[...] = jnp.where(m, y, prev)


def kernel(x, inds, w, b):
    xb = x.astype(jnp.bfloat16)
    wb = w.astype(jnp.bfloat16)
    inds2 = inds.astype(jnp.int32).reshape(_T, 1)
    return pl.pallas_call(
        _body,
        grid=(_CLASSES,),
        in_specs=[
            pl.BlockSpec((_T, 1), lambda c: (0, 0)),
            pl.BlockSpec((_T, _M), lambda c: (0, 0)),
            pl.BlockSpec((1, _M, _N), lambda c: (c, 0, 0)),
            pl.BlockSpec((1, 1, _N), lambda c: (c, 0, 0)),
        ],
        out_specs=pl.BlockSpec((_T, _N), lambda c: (0, 0)),
        out_shape=jax.ShapeDtypeStruct((_T, _N), jnp.float32),
    )(inds2, xb, wb, b)
